# Initial kernel scaffold; baseline (speedup 1.0000x reference)
import jax
import jax.numpy as jnp
from jax.experimental import pallas as pl


def kernel(x, edge_index, W_conv, b_conv, W1, b1, W2, b2, W3, b3):
    raise NotImplementedError("write your pallas kernel here")


# speedup vs baseline: 20.3203x; 20.3203x over previous
import functools

import jax
import jax.numpy as jnp
from jax import lax
from jax.experimental import pallas as pl
from jax.experimental.pallas import tpu as pltpu
from jax.experimental.pallas import tpu_sc as plsc

N = 10000
D = 128
NP = 10112
NC = 2
NS = 16
NW = NC * NS
ROWS_PT = 81
EDGES_PT = ROWS_PT * 128
E_PAD = NW * EDGES_PT
RPS = NP // NS
RB = 2000

_mesh = plsc.VectorSubcoreMesh(
    core_axis_name="c", subcore_axis_name="s", num_cores=NC, num_subcores=NS)


def _wid():
    return lax.axis_index("s") * NC + lax.axis_index("c")


def _sc_count_body(dst_hbm, hist_hbm, didx_v, hist_v):
    wid = _wid()

    def zero(i, _):
        hist_v[pl.ds(i * 16, 16)] = jnp.zeros((16,), jnp.float32)
        return 0
    lax.fori_loop(0, NP // 16, zero, 0)

    pltpu.sync_copy(dst_hbm.at[wid], didx_v)
    ones = jnp.ones((16,), jnp.float32)

    def row(r, _):
        for j in range(8):
            idx = didx_v[r, pl.ds(j * 16, 16)]
            plsc.addupdate_scatter(hist_v, [idx], ones)
        return 0
    lax.fori_loop(0, ROWS_PT, row, 0)

    pltpu.sync_copy(hist_v, hist_hbm.at[wid])


_sc_count = functools.partial(
    pl.kernel,
    out_type=jax.ShapeDtypeStruct((NW, NP), jnp.float32),
    mesh=_mesh,
    compiler_params=pltpu.CompilerParams(needs_layout_passes=False),
    scratch_types=[
        pltpu.VMEM((ROWS_PT, 128), jnp.int32),
        pltpu.VMEM((NP,), jnp.float32),
    ],
)(_sc_count_body)


def _sc_agg_body(y_hbm, src_hbm, dst_hbm, acc_hbm,
                 sidx_v, didx_v, rows_v, acc_sh, gsem):
    cid = lax.axis_index("c")
    sid = lax.axis_index("s")
    wid = _wid()

    def zrow(i, _):
        for j in range(8):
            rows_v[i, pl.ds(j * 16, 16)] = jnp.zeros((16,), jnp.float32)
        return 0
    lax.fori_loop(0, 128, zrow, 0)

    base = sid * RPS
    for k in range(4):
        pltpu.sync_copy(rows_v, acc_sh.at[pl.ds(base + k * 128, 128)])
    pltpu.sync_copy(rows_v.at[pl.ds(0, RPS - 512)],
                    acc_sh.at[pl.ds(base + 512, RPS - 512)])
    plsc.subcore_barrier()

    pltpu.sync_copy(src_hbm.at[wid], sidx_v)
    pltpu.sync_copy(dst_hbm.at[wid], didx_v)

    def chunk(j, _):
        pltpu.async_copy(y_hbm.at[sidx_v.at[j]], rows_v, gsem).wait()
        pltpu.sync_copy(rows_v, acc_sh.at[didx_v.at[j]], add=True)
        return 0
    lax.fori_loop(0, ROWS_PT, chunk, 0)

    plsc.subcore_barrier()
    pltpu.sync_copy(acc_sh.at[pl.ds(base, RPS)],
                    acc_hbm.at[pl.ds(cid * NP + base, RPS)])


_sc_agg = functools.partial(
    pl.kernel,
    out_type=jax.ShapeDtypeStruct((NC * NP, D), jnp.float32),
    mesh=_mesh,
    compiler_params=pltpu.CompilerParams(needs_layout_passes=False),
    scratch_types=[
        pltpu.VMEM((ROWS_PT, 128), jnp.int32),
        pltpu.VMEM((ROWS_PT, 128), jnp.int32),
        pltpu.VMEM((128, D), jnp.float32),
        pltpu.VMEM_SHARED((NP, D), jnp.float32),
        pltpu.SemaphoreType.DMA,
    ],
)(_sc_agg_body)


def _tc_y_body(x_ref, w_ref, h_ref, y_ref):
    xw = jnp.dot(x_ref[...], w_ref[...], preferred_element_type=jnp.float32)
    deg = jnp.sum(h_ref[0], axis=0)
    dinv = lax.rsqrt(deg)
    y_ref[...] = xw * dinv[:, None]


def _tc_y(x, w, hists):
    return pl.pallas_call(
        _tc_y_body,
        grid=(N // RB,),
        in_specs=[
            pl.BlockSpec((RB, D), lambda i: (i, 0)),
            pl.BlockSpec((D, D), lambda i: (0, 0)),
            pl.BlockSpec((1, NW, RB), lambda i: (i, 0, 0)),
        ],
        out_specs=pl.BlockSpec((RB, D), lambda i: (i, 0)),
        out_shape=jax.ShapeDtypeStruct((N, D), jnp.float32),
    )(x, w, hists)


def _tc_head_body(a0, a1, h, x_ref, bc, w1, b1, w2, b2, w3, b3, o_ref):
    deg = jnp.sum(h[0], axis=0)
    dinv = lax.rsqrt(deg)
    g = (a0[...] + a1[...]) * dinv[:, None] + bc[...]
    g = jnp.maximum(g, 0.0) + x_ref[...]
    z = jnp.maximum(
        jnp.dot(g, w1[...], preferred_element_type=jnp.float32) + b1[...], 0.0)
    z = jnp.maximum(
        jnp.dot(z, w2[...], preferred_element_type=jnp.float32) + b2[...], 0.0)
    o_ref[...] = jnp.dot(z, w3[...], preferred_element_type=jnp.float32) + b3[...]


def _tc_head(acc0, acc1, hists, x, bc, w1, b1, w2, b2, w3, b3):
    full = lambda r, c: pl.BlockSpec((r, c), lambda i: (0, 0))
    return pl.pallas_call(
        _tc_head_body,
        grid=(N // RB,),
        in_specs=[
            pl.BlockSpec((RB, D), lambda i: (i, 0)),
            pl.BlockSpec((RB, D), lambda i: (i, 0)),
            pl.BlockSpec((1, NW, RB), lambda i: (i, 0, 0)),
            pl.BlockSpec((RB, D), lambda i: (i, 0)),
            full(1, D), full(D, 32), full(1, 32),
            full(32, 32), full(1, 32), full(32, D), full(1, D),
        ],
        out_specs=pl.BlockSpec((RB, D), lambda i: (i, 0)),
        out_shape=jax.ShapeDtypeStruct((N, D), jnp.float32),
    )(acc0, acc1, hists, x, bc, w1, b1, w2, b2, w3, b3)


def kernel(x, edge_index, W_conv, b_conv, W1, b1, W2, b2, W3, b3):
    src = edge_index[0].astype(jnp.int32)
    dst = edge_index[1].astype(jnp.int32)
    e = src.shape[0]
    loops = jnp.arange(N, dtype=jnp.int32)
    npad = E_PAD - e - N
    src_all = jnp.concatenate(
        [src, loops, jnp.zeros((npad,), jnp.int32)]).reshape(NW, ROWS_PT, 128)
    dst_all = jnp.concatenate(
        [dst, loops, jnp.full((npad,), N, jnp.int32)]).reshape(NW, ROWS_PT, 128)

    hists = _sc_count(dst_all)
    hs3 = hists[:, :N].reshape(NW, N // RB, RB).transpose(1, 0, 2)
    y = _tc_y(x, W_conv, hs3)
    accs = _sc_agg(y, src_all, dst_all)
    acc0 = accs[:N]
    acc1 = accs[NP:NP + N]

    w3p = jnp.zeros((32, D), jnp.float32).at[:, :2].set(W3)
    b3p = jnp.zeros((1, D), jnp.float32).at[0, :2].set(b3)
    out = _tc_head(acc0, acc1, hs3, x, b_conv.reshape(1, D),
                   W1, b1.reshape(1, 32), W2, b2.reshape(1, 32), w3p, b3p)
    return out[:, :2]


# speedup vs baseline: 31.4073x; 1.5456x over previous
import functools

import jax
import jax.numpy as jnp
from jax import lax
from jax.experimental import pallas as pl
from jax.experimental.pallas import tpu as pltpu
from jax.experimental.pallas import tpu_sc as plsc

N = 10000
D = 128
NP = 10112
NC = 2
NS = 16
NW = NC * NS
ROWS_PT = 88
EDGES_PT = ROWS_PT * 128
E_PAD = NW * EDGES_PT
E_REAL = 320000 + N
RPS = NP // NS
RB = 2000

_mesh = plsc.VectorSubcoreMesh(
    core_axis_name="c", subcore_axis_name="s", num_cores=NC, num_subcores=NS)


def _wid():
    return lax.axis_index("s") * NC + lax.axis_index("c")


def _real_rows(wid):
    left = E_REAL - wid * EDGES_PT
    return jnp.clip((left + 127) // 128, 0, ROWS_PT)


def _sc_count_body(dst_hbm, hist_hbm, didx_v, hist_v):
    wid = _wid()

    def zero(i, _):
        hist_v[pl.ds(i * 16, 16)] = jnp.zeros((16,), jnp.float32)
        return 0
    lax.fori_loop(0, NP // 16, zero, 0)

    pltpu.sync_copy(dst_hbm.at[wid], didx_v)
    ones = jnp.ones((16,), jnp.float32)

    def row(r, _):
        for j in range(8):
            idx = didx_v[r, pl.ds(j * 16, 16)]
            plsc.addupdate_scatter(hist_v, [idx], ones)
        return 0
    lax.fori_loop(0, _real_rows(wid), row, 0)

    pltpu.sync_copy(hist_v, hist_hbm.at[wid])


_sc_count = functools.partial(
    pl.kernel,
    out_type=jax.ShapeDtypeStruct((NW, NP), jnp.float32),
    mesh=_mesh,
    compiler_params=pltpu.CompilerParams(needs_layout_passes=False),
    scratch_types=[
        pltpu.VMEM((ROWS_PT, 128), jnp.int32),
        pltpu.VMEM((NP,), jnp.float32),
    ],
)(_sc_count_body)


def _sc_agg_body(y_hbm, src_hbm, dst_hbm, acc_hbm,
                 sidx_v, didx_v, rows_v, acc_sh, gsem):
    cid = lax.axis_index("c")
    sid = lax.axis_index("s")
    wid = _wid()

    def zrow(i, _):
        for j in range(8):
            rows_v[0, i, pl.ds(j * 16, 16)] = jnp.zeros((16,), jnp.float32)
        return 0
    lax.fori_loop(0, 128, zrow, 0)

    base = sid * RPS
    for k in range(4):
        pltpu.sync_copy(rows_v.at[0], acc_sh.at[pl.ds(base + k * 128, 128)])
    pltpu.sync_copy(rows_v.at[0, pl.ds(0, RPS - 512)],
                    acc_sh.at[pl.ds(base + 512, RPS - 512)])
    plsc.subcore_barrier()

    pltpu.sync_copy(src_hbm.at[wid], sidx_v)
    rw = _real_rows(wid)

    @pl.when(rw > 0)
    def _():
        pltpu.async_copy(y_hbm.at[sidx_v.at[0]], rows_v.at[0], gsem)

    def group(g, _):
        gn = jnp.minimum(8, rw - g * 8)
        pltpu.sync_copy(dst_hbm.at[wid, pl.ds(g * 8, 8)], didx_v)

        def chunk(j, _):
            c = g * 8 + j
            buf = lax.rem(c, 2)
            pltpu.make_async_copy(
                y_hbm.at[sidx_v.at[c]], rows_v.at[buf], gsem).wait()

            @pl.when(c + 1 < rw)
            def _():
                pltpu.async_copy(
                    y_hbm.at[sidx_v.at[c + 1]], rows_v.at[1 - buf], gsem)
            pltpu.sync_copy(rows_v.at[buf], acc_sh.at[didx_v.at[j]], add=True)
            return 0
        lax.fori_loop(0, gn, chunk, 0)
        return 0
    lax.fori_loop(0, (rw + 7) // 8, group, 0)

    plsc.subcore_barrier()
    pltpu.sync_copy(acc_sh.at[pl.ds(base, RPS)],
                    acc_hbm.at[pl.ds(cid * NP + base, RPS)])


_sc_agg = functools.partial(
    pl.kernel,
    out_type=jax.ShapeDtypeStruct((NC * NP, D), jnp.float32),
    mesh=_mesh,
    compiler_params=pltpu.CompilerParams(needs_layout_passes=False),
    scratch_types=[
        pltpu.VMEM((ROWS_PT, 128), jnp.int32),
        pltpu.VMEM((8, 128), jnp.int32),
        pltpu.VMEM((2, 128, D), jnp.float32),
        pltpu.VMEM_SHARED((NP, D), jnp.float32),
        pltpu.SemaphoreType.DMA,
    ],
)(_sc_agg_body)


def _tc_y_body(x_ref, w_ref, h_ref, y_ref):
    xw = jnp.dot(x_ref[...], w_ref[...], preferred_element_type=jnp.float32)
    deg = jnp.sum(h_ref[0], axis=0)
    dinv = lax.rsqrt(deg)
    y_ref[...] = xw * dinv[:, None]


def _tc_y(x, w, hists):
    return pl.pallas_call(
        _tc_y_body,
        grid=(N // RB,),
        in_specs=[
            pl.BlockSpec((RB, D), lambda i: (i, 0)),
            pl.BlockSpec((D, D), lambda i: (0, 0)),
            pl.BlockSpec((1, NW, RB), lambda i: (i, 0, 0)),
        ],
        out_specs=pl.BlockSpec((RB, D), lambda i: (i, 0)),
        out_shape=jax.ShapeDtypeStruct((N, D), jnp.float32),
    )(x, w, hists)


def _tc_head_body(a0, a1, h, x_ref, bc, w1, b1, w2, b2, w3, b3, o_ref):
    deg = jnp.sum(h[0], axis=0)
    dinv = lax.rsqrt(deg)
    g = (a0[...] + a1[...]) * dinv[:, None] + bc[...]
    g = jnp.maximum(g, 0.0) + x_ref[...]
    z = jnp.maximum(
        jnp.dot(g, w1[...], preferred_element_type=jnp.float32) + b1[...], 0.0)
    z = jnp.maximum(
        jnp.dot(z, w2[...], preferred_element_type=jnp.float32) + b2[...], 0.0)
    o_ref[...] = jnp.dot(z, w3[...], preferred_element_type=jnp.float32) + b3[...]


def _tc_head(acc0, acc1, hists, x, bc, w1, b1, w2, b2, w3, b3):
    full = lambda r, c: pl.BlockSpec((r, c), lambda i: (0, 0))
    return pl.pallas_call(
        _tc_head_body,
        grid=(N // RB,),
        in_specs=[
            pl.BlockSpec((RB, D), lambda i: (i, 0)),
            pl.BlockSpec((RB, D), lambda i: (i, 0)),
            pl.BlockSpec((1, NW, RB), lambda i: (i, 0, 0)),
            pl.BlockSpec((RB, D), lambda i: (i, 0)),
            full(1, D), full(D, 32), full(1, 32),
            full(32, 32), full(1, 32), full(32, D), full(1, D),
        ],
        out_specs=pl.BlockSpec((RB, D), lambda i: (i, 0)),
        out_shape=jax.ShapeDtypeStruct((N, D), jnp.float32),
    )(acc0, acc1, hists, x, bc, w1, b1, w2, b2, w3, b3)


def kernel(x, edge_index, W_conv, b_conv, W1, b1, W2, b2, W3, b3):
    src = edge_index[0].astype(jnp.int32)
    dst = edge_index[1].astype(jnp.int32)
    e = src.shape[0]
    loops = jnp.arange(N, dtype=jnp.int32)
    npad = E_PAD - e - N
    src_all = jnp.concatenate(
        [src, loops, jnp.zeros((npad,), jnp.int32)]).reshape(NW, ROWS_PT, 128)
    dst_all = jnp.concatenate(
        [dst, loops, jnp.full((npad,), N, jnp.int32)]).reshape(NW, ROWS_PT, 128)

    hists = _sc_count(dst_all)
    hs3 = hists[:, :N].reshape(NW, N // RB, RB).transpose(1, 0, 2)
    y = _tc_y(x, W_conv, hs3)
    accs = _sc_agg(y, src_all, dst_all)
    acc0 = accs[:N]
    acc1 = accs[NP:NP + N]

    w3p = jnp.zeros((32, D), jnp.float32).at[:, :2].set(W3)
    b3p = jnp.zeros((1, D), jnp.float32).at[0, :2].set(b3)
    out = _tc_head(acc0, acc1, hs3, x, b_conv.reshape(1, D),
                   W1, b1.reshape(1, 32), W2, b2.reshape(1, 32), w3p, b3p)
    return out[:, :2]
